# Initial kernel scaffold; baseline (speedup 1.0000x reference)
import jax
import jax.numpy as jnp
from jax.experimental import pallas as pl


def kernel(node_data, edge_index, edge_data, lap_eigvec, atom_table, edge_table, graph_token, null_token, order_table, lap_W):
    raise NotImplementedError("write your pallas kernel here")


# speedup vs baseline: 70.6562x; 70.6562x over previous
import functools

import jax
import jax.numpy as jnp
from jax import lax
from jax.experimental import pallas as pl
from jax.experimental.pallas import tpu as pltpu
from jax.experimental.pallas import tpu_sc as plsc

_B = 16
_N = 1024
_E = 2048
_K = 32
_D = 1024
_T = _N + _E
_R = 2 + _T

_NC = 2
_NS = 16
_NW = _NC * _NS

_NPW = (_B * _N) // _NW
_EPW = (_B * _E) // _NW
_CN = 8
_CE = 16


def _tc_body(eig, src3, dst3, lapw, otab, gtok, ntok, out):
    def dot(a, b):
        return lax.dot(a, b, preferred_element_type=jnp.float32)

    w0 = lapw[0:_K, :]
    w1 = lapw[_K:2 * _K, :]
    w2 = lapw[2 * _K:3 * _K, :]
    eigb = eig[0]
    out[0, 0:1, :] = gtok[...]
    out[0, 1:2, :] = ntok[...]
    out[0, 2:2 + _N, :] = dot(eigb, w0 + w1 + w2) + otab[1:2, :]
    e0row = dot(eigb[0:1, :], w2)
    ce = 512
    for j in range(_E // ce):
        sj = src3[0, j * ce:(j + 1) * ce, :]
        dj = dst3[0, j * ce:(j + 1) * ce, :]
        iota = lax.broadcasted_iota(jnp.int32, (ce, _N), 1)
        ohs = (sj == iota).astype(jnp.float32)
        ohd = (dj == iota).astype(jnp.float32)
        lap = dot(dot(ohs, eigb), w0) + dot(dot(ohd, eigb), w1) + e0row
        orows = jnp.where(sj == dj, otab[1:2, :], otab[0:1, :])
        out[0, 2 + _N + j * ce:2 + _N + (j + 1) * ce, :] = lap + orows


def _tc_call(eig, src3, dst3, lapw, otab, gtok, ntok):
    return pl.pallas_call(
        _tc_body,
        grid=(_B,),
        in_specs=[
            pl.BlockSpec((1, _N, _K), lambda b: (b, 0, 0)),
            pl.BlockSpec((1, _E, 1), lambda b: (b, 0, 0)),
            pl.BlockSpec((1, _E, 1), lambda b: (b, 0, 0)),
            pl.BlockSpec((3 * _K, _D), lambda b: (0, 0)),
            pl.BlockSpec((2, _D), lambda b: (0, 0)),
            pl.BlockSpec((1, _D), lambda b: (0, 0)),
            pl.BlockSpec((1, _D), lambda b: (0, 0)),
        ],
        out_specs=pl.BlockSpec((1, _R, _D), lambda b: (b, 0, 0)),
        out_shape=jax.ShapeDtypeStruct((_B, _R, _D), jnp.float32),
    )(eig, src3, dst3, lapw, otab, gtok, ntok)


def _sc_body(nidx, eidx, dense, atab, etab, out,
             idx_n, idx_e, rows, acc, dns, sem):
    c = lax.axis_index("c")
    s = lax.axis_index("s")
    w = s * _NC + c
    b = w // 2
    half = w % 2
    brow = b * _R

    @pl.when(half == 0)
    def _():
        pltpu.sync_copy(dense.at[pl.ds(brow * _D, 2 * _D)],
                        dns.at[pl.ds(0, 2 * _D)])
        pltpu.sync_copy(dns.at[pl.ds(0, 2 * _D)],
                        out.at[pl.ds(brow * _D, 2 * _D)])

    def node_chunk(i, carry):
        g0 = w * _NPW + i * _CN
        row0 = brow + 2 + half * _NPW + i * _CN
        pltpu.sync_copy(nidx.at[pl.ds(g0 * 9, _CN * 9)], idx_n)
        cp = pltpu.async_copy(atab.at[idx_n], rows, sem)
        pltpu.sync_copy(dense.at[pl.ds(row0 * _D, _CN * _D)],
                        dns.at[pl.ds(0, _CN * _D)])
        cp.wait()

        def gbody(g, carry2):
            col = pl.ds(g * 16, 16)
            for t in range(_CN):
                pos = pl.ds(t * _D + g * 16, 16)
                a = dns[pos] + rows[t * 9, col]
                for f in range(1, 9):
                    a = a + rows[t * 9 + f, col]
                acc[pos] = a
            return carry2
        lax.fori_loop(0, _D // 16, gbody, 0)
        pltpu.sync_copy(acc.at[pl.ds(0, _CN * _D)],
                        out.at[pl.ds(row0 * _D, _CN * _D)])
        return carry
    lax.fori_loop(0, _NPW // _CN, node_chunk, 0)

    def edge_chunk(i, carry):
        g0 = w * _EPW + i * _CE
        row0 = brow + 2 + _N + half * _EPW + i * _CE
        pltpu.sync_copy(eidx.at[pl.ds(g0 * 3, _CE * 3)], idx_e)
        cp = pltpu.async_copy(etab.at[idx_e], rows.at[pl.ds(0, _CE * 3)], sem)
        pltpu.sync_copy(dense.at[pl.ds(row0 * _D, _CE * _D)], dns)
        cp.wait()

        def gbody(g, carry2):
            col = pl.ds(g * 16, 16)
            for t in range(_CE):
                pos = pl.ds(t * _D + g * 16, 16)
                acc[pos] = (dns[pos] + rows[t * 3, col]
                            + rows[t * 3 + 1, col] + rows[t * 3 + 2, col])
            return carry2
        lax.fori_loop(0, _D // 16, gbody, 0)
        pltpu.sync_copy(acc, out.at[pl.ds(row0 * _D, _CE * _D)])
        return carry
    lax.fori_loop(0, _EPW // _CE, edge_chunk, 0)


@functools.partial(jax.jit)
def _sc_call(nidx, eidx, dense, atab, etab):
    mesh = plsc.VectorSubcoreMesh(core_axis_name="c", subcore_axis_name="s")
    f = pl.kernel(
        _sc_body,
        mesh=mesh,
        out_type=jax.ShapeDtypeStruct((_B * _R * _D,), jnp.float32),
        scratch_types=[
            pltpu.VMEM((_CN * 9,), jnp.int32),
            pltpu.VMEM((_CE * 3,), jnp.int32),
            pltpu.VMEM((_CN * 9, _D), jnp.float32),
            pltpu.VMEM((_CE * _D,), jnp.float32),
            pltpu.VMEM((_CE * _D,), jnp.float32),
            pltpu.SemaphoreType.DMA,
        ],
    )
    return f(nidx, eidx, dense, atab, etab)


def kernel(node_data, edge_index, edge_data, lap_eigvec, atom_table, edge_table,
           graph_token, null_token, order_table, lap_W):
    nd = node_data.astype(jnp.int32)
    ed = edge_data.astype(jnp.int32)
    src = edge_index[0].astype(jnp.int32)
    dst = edge_index[1].astype(jnp.int32)

    dense = _tc_call(
        lap_eigvec.reshape(_B, _N, _K),
        src.reshape(_B, _E, 1), dst.reshape(_B, _E, 1),
        lap_W, order_table, graph_token, null_token)

    feat = _sc_call(
        nd.reshape(-1), ed.reshape(-1), dense.reshape(-1),
        atom_table, edge_table).reshape(_B, _R, _D)

    src_r = edge_index[0].reshape(_B, _E)
    dst_r = edge_index[1].reshape(_B, _E)
    node_idx_part = jnp.broadcast_to(
        jnp.arange(_N)[None, :, None], (_B, _N, 3))
    edge_idx_part = jnp.stack([src_r, dst_r, jnp.zeros_like(src_r)], axis=-1)
    padded_index = jnp.concatenate([node_idx_part, edge_idx_part], axis=1)
    padding_mask = jnp.zeros((_B, _R), dtype=bool)
    return feat, padding_mask, padded_index


# speedup vs baseline: 98.3231x; 1.3916x over previous
import functools

import jax
import jax.numpy as jnp
from jax import lax
from jax.experimental import pallas as pl
from jax.experimental.pallas import tpu as pltpu
from jax.experimental.pallas import tpu_sc as plsc

_B = 16
_N = 1024
_E = 2048
_K = 32
_D = 1024
_T = _N + _E
_R = 2 + _T

_NC = 2
_NS = 16
_NW = _NC * _NS

_NPW = (_B * _N) // _NW
_EPW = (_B * _E) // _NW
_CN = 4
_NF = 10
_CE = 8
_NCH_N = _NPW // _CN
_NCH_E = _EPW // _CE


def _sc_body(nidx, eidx, atab, etab, embn, embe,
             idxn_v, idxe_v, rows0, rows1, acc0, acc1,
             sg0, sg1, sw0, sw1):
    c = lax.axis_index("c")
    s = lax.axis_index("s")
    w = s * _NC + c
    rows = (rows0, rows1)
    acc = (acc0, acc1)
    sg = (sg0, sg1)
    sw = (sw0, sw1)

    pltpu.sync_copy(nidx.at[pl.ds(w * _NPW * _NF, _NPW * _NF)], idxn_v)
    pltpu.sync_copy(eidx.at[pl.ds(w * _EPW * 3, _EPW * 3)], idxe_v)

    nrow_n = _CN * _NF
    nrow_e = _CE * 3

    def n_gather(i, b):
        pltpu.async_copy(
            atab.at[idxn_v.at[pl.ds(i * nrow_n, nrow_n)]],
            rows[b].at[pl.ds(0, nrow_n)], sg[b])

    def n_wait(b):
        pltpu.make_async_copy(
            atab.at[pl.ds(0, nrow_n)],
            rows[b].at[pl.ds(0, nrow_n)], sg[b]).wait()

    def n_store(i, b):
        tok0 = w * _NPW + i * _CN
        pltpu.async_copy(acc[b].at[pl.ds(0, _CN)], embn.at[pl.ds(tok0, _CN)], sw[b])

    def n_store_wait(b):
        pltpu.make_async_copy(
            acc[b].at[pl.ds(0, _CN)], embn.at[pl.ds(0, _CN)], sw[b]).wait()

    n_gather(0, 0)

    def n_pair(p, carry):
        for b in range(2):
            i = 2 * p + b

            @pl.when(i + 1 < _NCH_N)
            def _():
                n_gather(i + 1, 1 - b)

            @pl.when(i >= 2)
            def _():
                n_store_wait(b)
            n_wait(b)

            def gbody(g, carry2):
                col = pl.ds(g * 16, 16)
                for t in range(_CN):
                    a = rows[b][t * _NF, col]
                    for f in range(1, 9):
                        a = a + rows[b][t * _NF + f, col]
                    acc[b][t, col] = a
                return carry2
            lax.fori_loop(0, _D // 16, gbody, 0)
            n_store(i, b)
        return carry
    lax.fori_loop(0, _NCH_N // 2, n_pair, 0)
    n_store_wait(0)
    n_store_wait(1)

    def e_gather(i, b):
        pltpu.async_copy(
            etab.at[idxe_v.at[pl.ds(i * nrow_e, nrow_e)]],
            rows[b].at[pl.ds(0, nrow_e)], sg[b])

    def e_wait(b):
        pltpu.make_async_copy(
            etab.at[pl.ds(0, nrow_e)],
            rows[b].at[pl.ds(0, nrow_e)], sg[b]).wait()

    def e_store(i, b):
        tok0 = w * _EPW + i * _CE
        pltpu.async_copy(acc[b].at[pl.ds(0, _CE)], embe.at[pl.ds(tok0, _CE)], sw[b])

    def e_store_wait(b):
        pltpu.make_async_copy(
            acc[b].at[pl.ds(0, _CE)], embe.at[pl.ds(0, _CE)], sw[b]).wait()

    e_gather(0, 0)

    def e_pair(p, carry):
        for b in range(2):
            i = 2 * p + b

            @pl.when(i + 1 < _NCH_E)
            def _():
                e_gather(i + 1, 1 - b)

            @pl.when(i >= 2)
            def _():
                e_store_wait(b)
            e_wait(b)

            def gbody(g, carry2):
                col = pl.ds(g * 16, 16)
                for t in range(_CE):
                    acc[b][t, col] = (rows[b][t * 3, col] + rows[b][t * 3 + 1, col]
                                      + rows[b][t * 3 + 2, col])
                return carry2
            lax.fori_loop(0, _D // 16, gbody, 0)
            e_store(i, b)
        return carry
    lax.fori_loop(0, _NCH_E // 2, e_pair, 0)
    e_store_wait(0)
    e_store_wait(1)


@functools.partial(jax.jit)
def _sc_call(nidx, eidx, atab, etab):
    mesh = plsc.VectorSubcoreMesh(core_axis_name="c", subcore_axis_name="s")
    f = pl.kernel(
        _sc_body,
        mesh=mesh,
        out_type=[
            jax.ShapeDtypeStruct((_B * _N, _D), jnp.float32),
            jax.ShapeDtypeStruct((_B * _E, _D), jnp.float32),
        ],
        scratch_types=[
            pltpu.VMEM((_NPW * _NF,), jnp.int32),
            pltpu.VMEM((_EPW * 3,), jnp.int32),
            pltpu.VMEM((_CN * _NF, _D), jnp.float32),
            pltpu.VMEM((_CN * _NF, _D), jnp.float32),
            pltpu.VMEM((_CE, _D), jnp.float32),
            pltpu.VMEM((_CE, _D), jnp.float32),
            pltpu.SemaphoreType.DMA,
            pltpu.SemaphoreType.DMA,
            pltpu.SemaphoreType.DMA,
            pltpu.SemaphoreType.DMA,
        ],
    )
    return f(nidx, eidx, atab, etab)


def _tc_body(embn, embe, eig, src3, dst3, lapw, otab, gtok, ntok, out):
    def dot(a, b):
        return lax.dot(a, b, preferred_element_type=jnp.float32)

    w0 = lapw[0:_K, :]
    w1 = lapw[_K:2 * _K, :]
    w2 = lapw[2 * _K:3 * _K, :]
    eigb = eig[0]
    out[0, 0:1, :] = gtok[...]
    out[0, 1:2, :] = ntok[...]
    wsum = w0 + w1 + w2
    cn = 512
    for j in range(_N // cn):
        nodelap = dot(eigb[j * cn:(j + 1) * cn, :], wsum)
        out[0, 2 + j * cn:2 + (j + 1) * cn, :] = (
            embn[0, j * cn:(j + 1) * cn, :] + nodelap + otab[1:2, :])
    e0row = dot(eigb[0:1, :], w2)
    ce = 512
    for j in range(_E // ce):
        sj = src3[0, j * ce:(j + 1) * ce, :]
        dj = dst3[0, j * ce:(j + 1) * ce, :]
        iota = lax.broadcasted_iota(jnp.int32, (ce, _N), 1)
        ohs = (sj == iota).astype(jnp.float32)
        ohd = (dj == iota).astype(jnp.float32)
        lap = dot(dot(ohs, eigb), w0) + dot(dot(ohd, eigb), w1) + e0row
        orows = jnp.where(sj == dj, otab[1:2, :], otab[0:1, :])
        out[0, 2 + _N + j * ce:2 + _N + (j + 1) * ce, :] = (
            embe[0, j * ce:(j + 1) * ce, :] + lap + orows)


def _tc_call(embn, embe, eig, src3, dst3, lapw, otab, gtok, ntok):
    return pl.pallas_call(
        _tc_body,
        grid=(_B,),
        in_specs=[
            pl.BlockSpec((1, _N, _D), lambda b: (b, 0, 0)),
            pl.BlockSpec((1, _E, _D), lambda b: (b, 0, 0)),
            pl.BlockSpec((1, _N, _K), lambda b: (b, 0, 0)),
            pl.BlockSpec((1, _E, 1), lambda b: (b, 0, 0)),
            pl.BlockSpec((1, _E, 1), lambda b: (b, 0, 0)),
            pl.BlockSpec((3 * _K, _D), lambda b: (0, 0)),
            pl.BlockSpec((2, _D), lambda b: (0, 0)),
            pl.BlockSpec((1, _D), lambda b: (0, 0)),
            pl.BlockSpec((1, _D), lambda b: (0, 0)),
        ],
        out_specs=pl.BlockSpec((1, _R, _D), lambda b: (b, 0, 0)),
        out_shape=jax.ShapeDtypeStruct((_B, _R, _D), jnp.float32),
    )(embn, embe, eig, src3, dst3, lapw, otab, gtok, ntok)


def kernel(node_data, edge_index, edge_data, lap_eigvec, atom_table, edge_table,
           graph_token, null_token, order_table, lap_W):
    nd = node_data.astype(jnp.int32)
    ed = edge_data.astype(jnp.int32)
    src = edge_index[0].astype(jnp.int32)
    dst = edge_index[1].astype(jnp.int32)

    nidx = jnp.pad(nd, ((0, 0), (0, _NF - 9))).reshape(-1)

    embn, embe = _sc_call(nidx, ed.reshape(-1), atom_table, edge_table)

    feat = _tc_call(
        embn.reshape(_B, _N, _D), embe.reshape(_B, _E, _D),
        lap_eigvec.reshape(_B, _N, _K),
        src.reshape(_B, _E, 1), dst.reshape(_B, _E, 1),
        lap_W, order_table, graph_token, null_token)

    src_r = edge_index[0].reshape(_B, _E)
    dst_r = edge_index[1].reshape(_B, _E)
    node_idx_part = jnp.broadcast_to(
        jnp.arange(_N)[None, :, None], (_B, _N, 3))
    edge_idx_part = jnp.stack([src_r, dst_r, jnp.zeros_like(src_r)], axis=-1)
    padded_index = jnp.concatenate([node_idx_part, edge_idx_part], axis=1)
    padding_mask = jnp.zeros((_B, _R), dtype=bool)
    return feat, padding_mask, padded_index


# speedup vs baseline: 110.1908x; 1.1207x over previous
import functools

import jax
import jax.numpy as jnp
from jax import lax
from jax.experimental import pallas as pl
from jax.experimental.pallas import tpu as pltpu
from jax.experimental.pallas import tpu_sc as plsc

_B = 16
_N = 1024
_E = 2048
_K = 32
_D = 1024
_V_EDGE = 1536
_T = _N + _E
_R = 2 + _T

_NC = 2
_NS = 16
_NW = _NC * _NS

_NPW = (_B * _N) // _NW
_CN = 4
_NF = 10
_NCH_N = _NPW // _CN


def _sc_body(nidx, atab, embn,
             idxn_v, rows0, rows1, acc0, acc1,
             sg0, sg1, sw0, sw1):
    c = lax.axis_index("c")
    s = lax.axis_index("s")
    w = s * _NC + c
    rows = (rows0, rows1)
    acc = (acc0, acc1)
    sg = (sg0, sg1)
    sw = (sw0, sw1)

    pltpu.sync_copy(nidx.at[pl.ds(w * _NPW * _NF, _NPW * _NF)], idxn_v)

    nrow_n = _CN * _NF

    def n_gather(i, b):
        pltpu.async_copy(
            atab.at[idxn_v.at[pl.ds(i * nrow_n, nrow_n)]],
            rows[b], sg[b])

    def n_wait(b):
        pltpu.make_async_copy(
            atab.at[pl.ds(0, nrow_n)], rows[b], sg[b]).wait()

    def n_store(i, b):
        tok0 = w * _NPW + i * _CN
        pltpu.async_copy(acc[b], embn.at[pl.ds(tok0, _CN)], sw[b])

    def n_store_wait(b):
        pltpu.make_async_copy(
            acc[b], embn.at[pl.ds(0, _CN)], sw[b]).wait()

    n_gather(0, 0)

    def n_pair(p, carry):
        for b in range(2):
            i = 2 * p + b

            @pl.when(i + 1 < _NCH_N)
            def _():
                n_gather(i + 1, 1 - b)

            @pl.when(i >= 2)
            def _():
                n_store_wait(b)
            n_wait(b)

            def gbody(g, carry2):
                col = pl.ds(g * 16, 16)
                for t in range(_CN):
                    a = rows[b][t * _NF, col]
                    for f in range(1, 9):
                        a = a + rows[b][t * _NF + f, col]
                    acc[b][t, col] = a
                return carry2
            lax.fori_loop(0, _D // 16, gbody, 0)
            n_store(i, b)
        return carry
    lax.fori_loop(0, _NCH_N // 2, n_pair, 0)
    n_store_wait(0)
    n_store_wait(1)


@functools.partial(jax.jit)
def _sc_call(nidx, atab):
    mesh = plsc.VectorSubcoreMesh(core_axis_name="c", subcore_axis_name="s")
    f = pl.kernel(
        _sc_body,
        mesh=mesh,
        out_type=jax.ShapeDtypeStruct((_B * _N, _D), jnp.float32),
        scratch_types=[
            pltpu.VMEM((_NPW * _NF,), jnp.int32),
            pltpu.VMEM((_CN * _NF, _D), jnp.float32),
            pltpu.VMEM((_CN * _NF, _D), jnp.float32),
            pltpu.VMEM((_CN, _D), jnp.float32),
            pltpu.VMEM((_CN, _D), jnp.float32),
            pltpu.SemaphoreType.DMA,
            pltpu.SemaphoreType.DMA,
            pltpu.SemaphoreType.DMA,
            pltpu.SemaphoreType.DMA,
        ],
    )
    return f(nidx, atab)


def _tca_body(eig, src3, dst3, ed0, ed1, ed2, lapw, otab, gtok, ntok, etab, out):
    def dot(a, b):
        return lax.dot(a, b, preferred_element_type=jnp.float32)

    w0 = lapw[0:_K, :]
    w1 = lapw[_K:2 * _K, :]
    w2 = lapw[2 * _K:3 * _K, :]
    eigb = eig[0]
    out[0, 0:1, :] = gtok[...]
    out[0, 1:2, :] = ntok[...]
    wsum = w0 + w1 + w2
    cn = 512
    for j in range(_N // cn):
        nodelap = dot(eigb[j * cn:(j + 1) * cn, :], wsum)
        out[0, 2 + j * cn:2 + (j + 1) * cn, :] = nodelap + otab[1:2, :]
    e0row = dot(eigb[0:1, :], w2)
    ce = 512
    for j in range(_E // ce):
        sl = slice(j * ce, (j + 1) * ce)
        sj = src3[0, sl, :]
        dj = dst3[0, sl, :]
        iota_n = lax.broadcasted_iota(jnp.int32, (ce, _N), 1)
        ohs = (sj == iota_n).astype(jnp.float32)
        ohd = (dj == iota_n).astype(jnp.float32)
        lap = dot(dot(ohs, eigb), w0) + dot(dot(ohd, eigb), w1) + e0row
        iota_e = lax.broadcasted_iota(jnp.int32, (ce, _V_EDGE), 1)
        cnt = ((ed0[0, sl, :] == iota_e).astype(jnp.float32)
               + (ed1[0, sl, :] == iota_e).astype(jnp.float32)
               + (ed2[0, sl, :] == iota_e).astype(jnp.float32))
        emb = dot(cnt, etab[...])
        orows = jnp.where(sj == dj, otab[1:2, :], otab[0:1, :])
        out[0, 2 + _N + j * ce:2 + _N + (j + 1) * ce, :] = emb + lap + orows


def _tca_call(eig, src3, dst3, ed0, ed1, ed2, lapw, otab, gtok, ntok, etab):
    return pl.pallas_call(
        _tca_body,
        grid=(_B,),
        in_specs=[
            pl.BlockSpec((1, _N, _K), lambda b: (b, 0, 0)),
            pl.BlockSpec((1, _E, 1), lambda b: (b, 0, 0)),
            pl.BlockSpec((1, _E, 1), lambda b: (b, 0, 0)),
            pl.BlockSpec((1, _E, 1), lambda b: (b, 0, 0)),
            pl.BlockSpec((1, _E, 1), lambda b: (b, 0, 0)),
            pl.BlockSpec((1, _E, 1), lambda b: (b, 0, 0)),
            pl.BlockSpec((3 * _K, _D), lambda b: (0, 0)),
            pl.BlockSpec((2, _D), lambda b: (0, 0)),
            pl.BlockSpec((1, _D), lambda b: (0, 0)),
            pl.BlockSpec((1, _D), lambda b: (0, 0)),
            pl.BlockSpec((_V_EDGE, _D), lambda b: (0, 0)),
        ],
        out_specs=pl.BlockSpec((1, _R, _D), lambda b: (b, 0, 0)),
        out_shape=jax.ShapeDtypeStruct((_B, _R, _D), jnp.float32),
    )(eig, src3, dst3, ed0, ed1, ed2, lapw, otab, gtok, ntok, etab)


_W_B = 1032


def _tcb_body(outa, embn, out):
    out[0, 0:2, :] = outa[0, 0:2, :]
    out[0, 2:2 + _N, :] = outa[0, 2:2 + _N, :] + embn[0]
    out[0, 2 + _N:_W_B, :] = outa[0, 2 + _N:_W_B, :]


def _tcb_call(outa, embn):
    return pl.pallas_call(
        _tcb_body,
        grid=(_B,),
        in_specs=[
            pl.BlockSpec((1, _W_B, _D), lambda b: (b, 0, 0)),
            pl.BlockSpec((1, _N, _D), lambda b: (b, 0, 0)),
        ],
        out_specs=pl.BlockSpec((1, _W_B, _D), lambda b: (b, 0, 0)),
        out_shape=jax.ShapeDtypeStruct((_B, _R, _D), jnp.float32),
        input_output_aliases={0: 0},
    )(outa, embn)


def kernel(node_data, edge_index, edge_data, lap_eigvec, atom_table, edge_table,
           graph_token, null_token, order_table, lap_W):
    nd = node_data.astype(jnp.int32)
    ed = edge_data.astype(jnp.int32)
    src = edge_index[0].astype(jnp.int32)
    dst = edge_index[1].astype(jnp.int32)

    nidx = jnp.pad(nd, ((0, 0), (0, _NF - 9))).reshape(-1)
    embn = _sc_call(nidx, atom_table)

    ed3 = ed.reshape(_B, _E, 3)
    outa = _tca_call(
        lap_eigvec.reshape(_B, _N, _K),
        src.reshape(_B, _E, 1), dst.reshape(_B, _E, 1),
        ed3[:, :, 0:1], ed3[:, :, 1:2], ed3[:, :, 2:3],
        lap_W, order_table, graph_token, null_token, edge_table)

    feat = _tcb_call(outa, embn.reshape(_B, _N, _D))

    src_r = edge_index[0].reshape(_B, _E)
    dst_r = edge_index[1].reshape(_B, _E)
    node_idx_part = jnp.broadcast_to(
        jnp.arange(_N)[None, :, None], (_B, _N, 3))
    edge_idx_part = jnp.stack([src_r, dst_r, jnp.zeros_like(src_r)], axis=-1)
    padded_index = jnp.concatenate([node_idx_part, edge_idx_part], axis=1)
    padding_mask = jnp.zeros((_B, _R), dtype=bool)
    return feat, padding_mask, padded_index


# speedup vs baseline: 221.9027x; 2.0138x over previous
import functools

import jax
import jax.numpy as jnp
from jax import lax
from jax.experimental import pallas as pl
from jax.experimental.pallas import tpu as pltpu
from jax.experimental.pallas import tpu_sc as plsc

_B = 16
_N = 1024
_E = 2048
_K = 32
_D = 1024
_V_EDGE = 1536
_T = _N + _E
_R = 2 + _T

_NC = 2
_NS = 16
_NW = _NC * _NS

_NPW = (_B * _N) // _NW
_CN = 4
_NF = 10
_NCH_N = _NPW // _CN


def _sc_body(nidx, atab, embn,
             idxn_v, rows0, rows1, acc0, acc1,
             sg0, sg1, sw0, sw1):
    c = lax.axis_index("c")
    s = lax.axis_index("s")
    w = s * _NC + c
    rows = (rows0, rows1)
    acc = (acc0, acc1)
    sg = (sg0, sg1)
    sw = (sw0, sw1)

    pltpu.sync_copy(nidx.at[pl.ds(w * _NPW * _NF, _NPW * _NF)], idxn_v)

    nrow_n = _CN * _NF

    def n_gather(i, b):
        pltpu.async_copy(
            atab.at[idxn_v.at[pl.ds(i * nrow_n, nrow_n)]],
            rows[b], sg[b])

    def n_wait(b):
        pltpu.make_async_copy(
            atab.at[pl.ds(0, nrow_n)], rows[b], sg[b]).wait()

    def n_store8(q, half):
        tok0 = w * _NPW + (4 * q + 2 * half) * _CN
        pltpu.async_copy(acc[half], embn.at[pl.ds(tok0, 2 * _CN)], sw[half])

    def n_store_wait(half):
        pltpu.make_async_copy(
            acc[half], embn.at[pl.ds(0, 2 * _CN)], sw[half]).wait()

    n_gather(0, 0)

    def n_quad(q, carry):
        for half in range(2):
            for sub in range(2):
                i = 4 * q + 2 * half + sub
                b = sub

                @pl.when(i + 1 < _NCH_N)
                def _():
                    n_gather(i + 1, 1 - b)

                if sub == 0:
                    @pl.when(q >= 1)
                    def _():
                        n_store_wait(half)
                n_wait(b)

                def gbody(g, carry2):
                    col = pl.ds(g * 16, 16)
                    for t in range(_CN):
                        a = rows[b][t * _NF, col]
                        for f in range(1, 9):
                            a = a + rows[b][t * _NF + f, col]
                        acc[half][sub * _CN + t, col] = a
                    return carry2
                lax.fori_loop(0, _D // 16, gbody, 0)
            n_store8(q, half)
        return carry
    lax.fori_loop(0, _NCH_N // 4, n_quad, 0)
    n_store_wait(0)
    n_store_wait(1)


@functools.partial(jax.jit)
def _sc_call(nidx, atab):
    mesh = plsc.VectorSubcoreMesh(core_axis_name="c", subcore_axis_name="s")
    f = pl.kernel(
        _sc_body,
        mesh=mesh,
        out_type=jax.ShapeDtypeStruct((_B * _N, _D), jnp.float32),
        scratch_types=[
            pltpu.VMEM((_NPW * _NF,), jnp.int32),
            pltpu.VMEM((_CN * _NF, _D), jnp.float32),
            pltpu.VMEM((_CN * _NF, _D), jnp.float32),
            pltpu.VMEM((2 * _CN, _D), jnp.float32),
            pltpu.VMEM((2 * _CN, _D), jnp.float32),
            pltpu.SemaphoreType.DMA,
            pltpu.SemaphoreType.DMA,
            pltpu.SemaphoreType.DMA,
            pltpu.SemaphoreType.DMA,
        ],
    )
    return f(nidx, atab)


def _tca_body(eig, src3, dst3, ed0, ed1, ed2, lapw, otab, gtok, ntok, etab, out):
    def dot(a, b):
        return lax.dot(a, b, preferred_element_type=jnp.float32)

    w0 = lapw[0:_K, :]
    w1 = lapw[_K:2 * _K, :]
    w2 = lapw[2 * _K:3 * _K, :]
    eigb = eig[0]
    out[0, 0:1, :] = gtok[...]
    out[0, 1:2, :] = ntok[...]
    wsum = w0 + w1 + w2
    cn = 512
    for j in range(_N // cn):
        nodelap = dot(eigb[j * cn:(j + 1) * cn, :], wsum)
        out[0, 2 + j * cn:2 + (j + 1) * cn, :] = nodelap + otab[1:2, :]
    e0row = dot(eigb[0:1, :], w2)
    ce = 512
    for j in range(_E // ce):
        sl = slice(j * ce, (j + 1) * ce)
        sj = src3[0, sl, :]
        dj = dst3[0, sl, :]
        iota_n = lax.broadcasted_iota(jnp.int32, (ce, _N), 1)
        ohs = (sj == iota_n).astype(jnp.float32)
        ohd = (dj == iota_n).astype(jnp.float32)
        lap = dot(dot(ohs, eigb), w0) + dot(dot(ohd, eigb), w1) + e0row
        iota_e = lax.broadcasted_iota(jnp.int32, (ce, _V_EDGE), 1)
        cnt = ((ed0[0, sl, :] == iota_e).astype(jnp.float32)
               + (ed1[0, sl, :] == iota_e).astype(jnp.float32)
               + (ed2[0, sl, :] == iota_e).astype(jnp.float32))
        emb = dot(cnt, etab[...])
        orows = jnp.where(sj == dj, otab[1:2, :], otab[0:1, :])
        out[0, 2 + _N + j * ce:2 + _N + (j + 1) * ce, :] = emb + lap + orows


def _tca_call(eig, src3, dst3, ed0, ed1, ed2, lapw, otab, gtok, ntok, etab):
    return pl.pallas_call(
        _tca_body,
        grid=(_B,),
        in_specs=[
            pl.BlockSpec((1, _N, _K), lambda b: (b, 0, 0)),
            pl.BlockSpec((1, _E, 1), lambda b: (b, 0, 0)),
            pl.BlockSpec((1, _E, 1), lambda b: (b, 0, 0)),
            pl.BlockSpec((1, _E, 1), lambda b: (b, 0, 0)),
            pl.BlockSpec((1, _E, 1), lambda b: (b, 0, 0)),
            pl.BlockSpec((1, _E, 1), lambda b: (b, 0, 0)),
            pl.BlockSpec((3 * _K, _D), lambda b: (0, 0)),
            pl.BlockSpec((2, _D), lambda b: (0, 0)),
            pl.BlockSpec((1, _D), lambda b: (0, 0)),
            pl.BlockSpec((1, _D), lambda b: (0, 0)),
            pl.BlockSpec((_V_EDGE, _D), lambda b: (0, 0)),
        ],
        out_specs=pl.BlockSpec((1, _R, _D), lambda b: (b, 0, 0)),
        out_shape=jax.ShapeDtypeStruct((_B, _R, _D), jnp.float32),
    )(eig, src3, dst3, ed0, ed1, ed2, lapw, otab, gtok, ntok, etab)


_W_B = 1032


def _tcb_body(outa, embn, out):
    out[0, 0:2, :] = outa[0, 0:2, :]
    out[0, 2:2 + _N, :] = outa[0, 2:2 + _N, :] + embn[0]
    out[0, 2 + _N:_W_B, :] = outa[0, 2 + _N:_W_B, :]


def _tcb_call(outa, embn):
    return pl.pallas_call(
        _tcb_body,
        grid=(_B,),
        in_specs=[
            pl.BlockSpec((1, _W_B, _D), lambda b: (b, 0, 0)),
            pl.BlockSpec((1, _N, _D), lambda b: (b, 0, 0)),
        ],
        out_specs=pl.BlockSpec((1, _W_B, _D), lambda b: (b, 0, 0)),
        out_shape=jax.ShapeDtypeStruct((_B, _R, _D), jnp.float32),
        input_output_aliases={0: 0},
    )(outa, embn)


def kernel(node_data, edge_index, edge_data, lap_eigvec, atom_table, edge_table,
           graph_token, null_token, order_table, lap_W):
    nd = node_data.astype(jnp.int32)
    ed = edge_data.astype(jnp.int32)
    src = edge_index[0].astype(jnp.int32)
    dst = edge_index[1].astype(jnp.int32)

    pad_col = (jnp.arange(_B * _N, dtype=jnp.int32) * 9) % 4608
    nidx = jnp.concatenate([nd, pad_col[:, None]], axis=1).reshape(-1)
    embn = _sc_call(nidx, atom_table)

    ed3 = ed.reshape(_B, _E, 3)
    outa = _tca_call(
        lap_eigvec.reshape(_B, _N, _K),
        src.reshape(_B, _E, 1), dst.reshape(_B, _E, 1),
        ed3[:, :, 0:1], ed3[:, :, 1:2], ed3[:, :, 2:3],
        lap_W, order_table, graph_token, null_token, edge_table)

    feat = _tcb_call(outa, embn.reshape(_B, _N, _D))

    src_r = edge_index[0].reshape(_B, _E)
    dst_r = edge_index[1].reshape(_B, _E)
    node_idx_part = jnp.broadcast_to(
        jnp.arange(_N)[None, :, None], (_B, _N, 3))
    edge_idx_part = jnp.stack([src_r, dst_r, jnp.zeros_like(src_r)], axis=-1)
    padded_index = jnp.concatenate([node_idx_part, edge_idx_part], axis=1)
    padding_mask = jnp.zeros((_B, _R), dtype=bool)
    return feat, padding_mask, padded_index
